# baseline (speedup 1.0000x reference)
import jax
import jax.numpy as jnp
from jax.experimental import pallas as pl
from jax.experimental.pallas import tpu as pltpu

N = 10000
NB = 25
BM = N // NB


def _gcn_body(x_ref, adj_ref, w1_ref, b1_ref, w2_ref, b2_ref, out_ref,
              support_ref, hw_ref):
    i = pl.program_id(0)

    @pl.when(i == 0)
    def _():
        support_ref[:, :] = jnp.dot(
            x_ref[:, :], w1_ref[:, :], preferred_element_type=jnp.float32)

    @pl.when(i < NB)
    def _():
        s1 = jnp.dot(adj_ref[:, :], support_ref[:, :],
                     preferred_element_type=jnp.float32)
        h = jnp.maximum(s1 + b1_ref[0, :], 0.0)
        row = i * BM
        hw_ref[pl.ds(row, BM), :] = jnp.dot(
            h, w2_ref[:, :], preferred_element_type=jnp.float32)

    @pl.when(i >= NB)
    def _():
        s2 = jnp.dot(adj_ref[:, :], hw_ref[:, :],
                     preferred_element_type=jnp.float32)
        z = s2 + b2_ref[0, :]
        m = jnp.max(z, axis=1, keepdims=True)
        lse = m + jnp.log(jnp.sum(jnp.exp(z - m), axis=1, keepdims=True))
        out_ref[:, :] = z - lse


@jax.jit
def kernel(x, adj, W1, b1, W2, b2):
    b1 = b1.reshape(1, -1)
    b2 = b2.reshape(1, -1)
    nhid = W1.shape[1]
    nclass = W2.shape[1]

    out = pl.pallas_call(
        _gcn_body,
        grid=(2 * NB,),
        in_specs=[
            pl.BlockSpec((N, x.shape[1]), lambda i: (0, 0)),
            pl.BlockSpec((BM, N), lambda i: (jax.lax.rem(i, NB), 0)),
            pl.BlockSpec(W1.shape, lambda i: (0, 0)),
            pl.BlockSpec((1, nhid), lambda i: (0, 0)),
            pl.BlockSpec(W2.shape, lambda i: (0, 0)),
            pl.BlockSpec((1, nclass), lambda i: (0, 0)),
        ],
        out_specs=pl.BlockSpec(
            (BM, nclass), lambda i: (jnp.maximum(i - NB, 0), 0)),
        out_shape=jax.ShapeDtypeStruct((N, nclass), jnp.float32),
        scratch_shapes=[
            pltpu.VMEM((N, nhid), jnp.float32),
            pltpu.VMEM((N, nclass), jnp.float32),
        ],
    )(x, adj, W1, b1, W2, b2)
    return out
